# baseline (speedup 1.0000x reference)
import functools

import jax
import jax.numpy as jnp
from jax import lax
from jax.experimental import pallas as pl
from jax.experimental.pallas import tpu as pltpu
from jax.experimental.pallas import tpu_sc as plsc

D = 1024
E = 8
KTOP = 2
NTOK = 2048
NSLOT = NTOK * KTOP
B = 256
GMAX = NSLOT // B + E
R = GMAX * B
S = 512
C = NSLOT // S

NC, NS = 2, 16
NW = NC * NS
TPW = NTOK // NW


def _gating_body(xf_ref, wgt_ref, bg_ref, dest_ref, w2_ref, be_ref):
    xf = xf_ref[...]
    logits = jnp.dot(xf, wgt_ref[...], preferred_element_type=jnp.float32)
    logits = logits + bg_ref[...]
    m = jnp.max(logits, axis=-1, keepdims=True)
    p = jnp.exp(logits - m)
    p = p / jnp.sum(p, axis=-1, keepdims=True)

    lane = lax.broadcasted_iota(jnp.int32, (NTOK, E), 1)
    m0 = jnp.max(p, axis=-1, keepdims=True)
    i0 = jnp.min(jnp.where(p == m0, lane, E + 1), axis=-1, keepdims=True)
    p2 = jnp.where(lane == i0, -1.0, p)
    m1 = jnp.max(p2, axis=-1, keepdims=True)
    i1 = jnp.min(jnp.where(p2 == m1, lane, E + 1), axis=-1, keepdims=True)

    e1 = jnp.exp(m1 - m0)
    w0 = 1.0 / (1.0 + e1)
    w1 = e1 / (1.0 + e1)
    w2_ref[...] = jnp.concatenate([w0, w1], axis=1)

    pi_col = jnp.concatenate([i0, i1], axis=0).astype(jnp.float32)

    def fmask(shape, fn):
        a = lax.broadcasted_iota(jnp.int32, shape, 0)
        b = lax.broadcasted_iota(jnp.int32, shape, 1)
        return fn(a, b).astype(jnp.float32)

    A = fmask((S, NSLOT), lambda i, s: s % S == i)
    cmask = fmask((NSLOT, C), lambda s, c: s // S == c)
    Pmat = pi_col * cmask
    PT = jnp.dot(A, Pmat, preferred_element_type=jnp.float32)

    Rep = fmask((C, C * E), lambda c, j: j // E == c)
    PTrep = jnp.dot(PT, Rep, preferred_element_type=jnp.float32)
    jmod = (lax.broadcasted_iota(jnp.int32, (S, C * E), 1) % E
            ).astype(jnp.float32)
    M = jnp.where(PTrep == jmod, 1.0, 0.0)

    Lexc = fmask((S, S), lambda i, j: j < i)
    cum = jnp.dot(Lexc, M, preferred_element_type=jnp.float32)

    totals = jnp.dot(jnp.ones((1, S), jnp.float32), M,
                     preferred_element_type=jnp.float32)

    Q = fmask((C * E, C * E),
              lambda i, j: (i % E == j % E) & (i // E < j // E))
    offs = jnp.dot(totals, Q, preferred_element_type=jnp.float32,
                   precision=lax.Precision.HIGHEST)

    Qg = fmask((C * E, E), lambda i, e: i % E == e)
    g_row = jnp.dot(totals, Qg, preferred_element_type=jnp.float32,
                   precision=lax.Precision.HIGHEST)
    gpad = jnp.floor((g_row + (B - 1)) * (1.0 / B)) * B
    Tri8 = fmask((E, E), lambda i, j: i < j)
    padoff = jnp.dot(gpad, Tri8, preferred_element_type=jnp.float32,
                   precision=lax.Precision.HIGHEST)

    Rep2 = fmask((E, C * E), lambda e, j: j % E == e)
    padoff_col = jnp.dot(padoff, Rep2, preferred_element_type=jnp.float32,
                   precision=lax.Precision.HIGHEST)

    base = offs + padoff_col
    Dcol = M * (cum + base)
    Sum8 = fmask((C * E, C), lambda j, c: j // E == c)
    Dmat = jnp.dot(Dcol, Sum8, preferred_element_type=jnp.float32,
                   precision=lax.Precision.HIGHEST)
    dest_ref[...] = Dmat.astype(jnp.int32)

    padoffT = jnp.transpose(padoff)
    blk = (lax.broadcasted_iota(jnp.int32, (1, 128), 1) * B
           ).astype(jnp.float32)
    cnt = jnp.sum(jnp.where(padoffT <= blk, 1.0, 0.0), axis=0,
                  keepdims=True)
    be = jnp.clip(cnt - 1.0, 0.0, E - 1.0)
    total_pad = padoff[0:1, E - 1:E] + gpad[0:1, E - 1:E]
    act = jnp.where(blk < total_pad, 1.0, 0.0)
    be_ref[...] = jnp.concatenate([be, act], axis=0).astype(jnp.int32)


def _gating_call(xf, wgt, bg2):
    return pl.pallas_call(
        _gating_body,
        out_shape=(
            jax.ShapeDtypeStruct((S, C), jnp.int32),
            jax.ShapeDtypeStruct((NTOK, 2), jnp.float32),
            jax.ShapeDtypeStruct((2, 128), jnp.int32),
        ),
    )(xf, wgt, bg2)


def _dispatch_body(xf_hbm, da_hbm, db_hbm, xs_hbm, xrows, ia, ib, sem):
    wid = lax.axis_index("s") * NC + lax.axis_index("c")
    base = wid * TPW
    pltpu.sync_copy(xf_hbm.at[pl.ds(base, TPW), :], xrows)
    pltpu.sync_copy(da_hbm.at[pl.ds(base, TPW)], ia)
    pltpu.sync_copy(db_hbm.at[pl.ds(base, TPW)], ib)
    c1 = pltpu.async_copy(xrows, xs_hbm.at[ia], sem)
    c2 = pltpu.async_copy(xrows, xs_hbm.at[ib], sem)
    c1.wait()
    c2.wait()


def _dispatch_call(xf, da, db):
    mesh = plsc.VectorSubcoreMesh(core_axis_name="c", subcore_axis_name="s")
    f = pl.kernel(
        _dispatch_body,
        out_type=jax.ShapeDtypeStruct((R, D), jnp.float32),
        mesh=mesh,
        scratch_types=[
            pltpu.VMEM((TPW, D), jnp.float32),
            pltpu.VMEM((TPW,), jnp.int32),
            pltpu.VMEM((TPW,), jnp.int32),
            pltpu.SemaphoreType.DMA,
        ],
    )
    return f(xf, da, db)


_NT = (((1,), (1,)), ((), ()))


def _ffn_body(be_ref, act_ref, xs_ref, w1_ref, b1_ref, w2_ref, b2_ref,
              ys_ref):
    g = pl.program_id(0)

    @pl.when(act_ref[g] > 0)
    def _():
        x = xs_ref[...].astype(jnp.bfloat16)
        h = lax.dot_general(x, w1_ref[0], _NT,
                            preferred_element_type=jnp.float32)
        h = jnp.maximum(h + b1_ref[0], 0.0)
        y = lax.dot_general(h.astype(jnp.bfloat16), w2_ref[0], _NT,
                            preferred_element_type=jnp.float32)
        ys_ref[...] = y + b2_ref[0]


def _ffn_call(be, act, xs, w1, b1r, w2, b2r):
    grid_spec = pltpu.PrefetchScalarGridSpec(
        num_scalar_prefetch=2,
        grid=(GMAX,),
        in_specs=[
            pl.BlockSpec((B, D), lambda g, be, act: (g, 0)),
            pl.BlockSpec((1, D, D), lambda g, be, act: (be[g], 0, 0)),
            pl.BlockSpec((1, 1, D), lambda g, be, act: (be[g], 0, 0)),
            pl.BlockSpec((1, D, D), lambda g, be, act: (be[g], 0, 0)),
            pl.BlockSpec((1, 1, D), lambda g, be, act: (be[g], 0, 0)),
        ],
        out_specs=pl.BlockSpec((B, D), lambda g, be, act: (g, 0)),
    )
    return pl.pallas_call(
        _ffn_body,
        grid_spec=grid_spec,
        out_shape=jax.ShapeDtypeStruct((R, D), jnp.float32),
    )(be, act, xs, w1, b1r, w2, b2r)


CH = TPW // 2


def _combine_body(ys_hbm, da_hbm, db_hbm, ya_hbm, yb_hbm, bufa, bufb, ia, ib,
                  sem):
    wid = lax.axis_index("s") * NC + lax.axis_index("c")
    base = wid * TPW
    for chunk in range(2):
        tbase = base + chunk * CH
        pltpu.sync_copy(da_hbm.at[pl.ds(tbase, CH)], ia)
        pltpu.sync_copy(db_hbm.at[pl.ds(tbase, CH)], ib)
        ca = pltpu.async_copy(ys_hbm.at[ia], bufa, sem)
        cb = pltpu.async_copy(ys_hbm.at[ib], bufb, sem)
        ca.wait()
        cb.wait()
        pltpu.sync_copy(bufa, ya_hbm.at[pl.ds(tbase, CH), :])
        pltpu.sync_copy(bufb, yb_hbm.at[pl.ds(tbase, CH), :])


def _combine_call(ys, da, db):
    mesh = plsc.VectorSubcoreMesh(core_axis_name="c", subcore_axis_name="s")
    f = pl.kernel(
        _combine_body,
        out_type=(
            jax.ShapeDtypeStruct((NTOK, D), jnp.float32),
            jax.ShapeDtypeStruct((NTOK, D), jnp.float32),
        ),
        mesh=mesh,
        scratch_types=[
            pltpu.VMEM((CH, D), jnp.float32),
            pltpu.VMEM((CH, D), jnp.float32),
            pltpu.VMEM((CH,), jnp.int32),
            pltpu.VMEM((CH,), jnp.int32),
            pltpu.SemaphoreType.DMA,
        ],
    )
    return f(ys, da, db)


def _add_body(a_ref, b_ref, w_ref, o_ref):
    wv = w_ref[...]
    o_ref[...] = a_ref[...] * wv[:, 0:1] + b_ref[...] * wv[:, 1:2]


def _add_call(a, b, w2col):
    return pl.pallas_call(
        _add_body,
        grid=(4,),
        in_specs=[
            pl.BlockSpec((NTOK // 4, D), lambda g: (g, 0)),
            pl.BlockSpec((NTOK // 4, D), lambda g: (g, 0)),
            pl.BlockSpec((NTOK // 4, 2), lambda g: (g, 0)),
        ],
        out_specs=pl.BlockSpec((NTOK // 4, D), lambda g: (g, 0)),
        out_shape=jax.ShapeDtypeStruct((NTOK, D), jnp.float32),
    )(a, b, w2col)


def kernel(x, Wg, bg, W1, b1, W2, b2):
    xf = x.reshape(-1, D)
    dest_mat, w2col, be_row = _gating_call(xf, Wg.T, bg.reshape(1, E))

    d = jnp.transpose(dest_mat).reshape(-1)
    da, db = d[:NTOK], d[NTOK:]
    be = be_row[0, :GMAX]
    act = be_row[1, :GMAX]

    xs = _dispatch_call(xf, da, db)
    ys = _ffn_call(be, act, xs, W1.astype(jnp.bfloat16),
                   b1.reshape(E, 1, D), W2.astype(jnp.bfloat16),
                   b2.reshape(E, 1, D))
    ya, yb = _combine_call(ys, da, db)
    y = _add_call(ya, yb, w2col)
    return y.reshape(x.shape)


# speedup vs baseline: 1.1776x; 1.1776x over previous
import functools

import jax
import jax.numpy as jnp
from jax import lax
from jax.experimental import pallas as pl
from jax.experimental.pallas import tpu as pltpu
from jax.experimental.pallas import tpu_sc as plsc

D = 1024
E = 8
KTOP = 2
NTOK = 2048
NSLOT = NTOK * KTOP
B = 256
GMAX = NSLOT // B + E
R = GMAX * B
S = 512
C = NSLOT // S

NC, NS = 2, 16
NW = NC * NS
TPW = NTOK // NW


def _gating_body(xf_ref, wgt_ref, bg_ref, dest_ref, w2_ref, be_ref):
    xf = xf_ref[...]
    logits = jnp.dot(xf, wgt_ref[...], preferred_element_type=jnp.float32)
    logits = logits + bg_ref[...]
    m = jnp.max(logits, axis=-1, keepdims=True)
    p = jnp.exp(logits - m)
    p = p / jnp.sum(p, axis=-1, keepdims=True)

    lane = lax.broadcasted_iota(jnp.int32, (NTOK, E), 1)
    m0 = jnp.max(p, axis=-1, keepdims=True)
    i0 = jnp.min(jnp.where(p == m0, lane, E + 1), axis=-1, keepdims=True)
    p2 = jnp.where(lane == i0, -1.0, p)
    m1 = jnp.max(p2, axis=-1, keepdims=True)
    i1 = jnp.min(jnp.where(p2 == m1, lane, E + 1), axis=-1, keepdims=True)

    e1 = jnp.exp(m1 - m0)
    w0 = 1.0 / (1.0 + e1)
    w1 = e1 / (1.0 + e1)
    w2_ref[...] = jnp.concatenate([w0, w1], axis=1)

    pi_col = jnp.concatenate([i0, i1], axis=0).astype(jnp.float32)

    def fmask(shape, fn):
        a = lax.broadcasted_iota(jnp.int32, shape, 0)
        b = lax.broadcasted_iota(jnp.int32, shape, 1)
        return fn(a, b).astype(jnp.float32)

    A = fmask((S, NSLOT), lambda i, s: s % S == i)
    cmask = fmask((NSLOT, C), lambda s, c: s // S == c)
    Pmat = pi_col * cmask
    PT = jnp.dot(A, Pmat, preferred_element_type=jnp.float32)

    Rep = fmask((C, C * E), lambda c, j: j // E == c)
    PTrep = jnp.dot(PT, Rep, preferred_element_type=jnp.float32)
    jmod = (lax.broadcasted_iota(jnp.int32, (S, C * E), 1) % E
            ).astype(jnp.float32)
    M = jnp.where(PTrep == jmod, 1.0, 0.0)

    Lexc = fmask((S, S), lambda i, j: j < i)
    cum = jnp.dot(Lexc, M, preferred_element_type=jnp.float32)

    totals = jnp.dot(jnp.ones((1, S), jnp.float32), M,
                     preferred_element_type=jnp.float32)

    Q = fmask((C * E, C * E),
              lambda i, j: (i % E == j % E) & (i // E < j // E))
    offs = jnp.dot(totals, Q, preferred_element_type=jnp.float32,
                   precision=lax.Precision.HIGHEST)

    Qg = fmask((C * E, E), lambda i, e: i % E == e)
    g_row = jnp.dot(totals, Qg, preferred_element_type=jnp.float32,
                   precision=lax.Precision.HIGHEST)
    gpad = jnp.floor((g_row + (B - 1)) * (1.0 / B)) * B
    Tri8 = fmask((E, E), lambda i, j: i < j)
    padoff = jnp.dot(gpad, Tri8, preferred_element_type=jnp.float32,
                   precision=lax.Precision.HIGHEST)

    Rep2 = fmask((E, C * E), lambda e, j: j % E == e)
    padoff_col = jnp.dot(padoff, Rep2, preferred_element_type=jnp.float32,
                   precision=lax.Precision.HIGHEST)

    base = offs + padoff_col
    Dcol = M * (cum + base)
    Sum8 = fmask((C * E, C), lambda j, c: j // E == c)
    Dmat = jnp.dot(Dcol, Sum8, preferred_element_type=jnp.float32,
                   precision=lax.Precision.HIGHEST)
    dest_ref[...] = Dmat.astype(jnp.int32)

    padoffT = jnp.transpose(padoff)
    blk = (lax.broadcasted_iota(jnp.int32, (1, 128), 1) * B
           ).astype(jnp.float32)
    cnt = jnp.sum(jnp.where(padoffT <= blk, 1.0, 0.0), axis=0,
                  keepdims=True)
    be = jnp.clip(cnt - 1.0, 0.0, E - 1.0)
    total_pad = padoff[0:1, E - 1:E] + gpad[0:1, E - 1:E]
    act = jnp.where(blk < total_pad, 1.0, 0.0)
    be_ref[...] = jnp.concatenate([be, act], axis=0).astype(jnp.int32)


def _gating_call(xf, wgt, bg2):
    return pl.pallas_call(
        _gating_body,
        out_shape=(
            jax.ShapeDtypeStruct((S, C), jnp.int32),
            jax.ShapeDtypeStruct((NTOK, 2), jnp.float32),
            jax.ShapeDtypeStruct((2, 128), jnp.int32),
        ),
    )(xf, wgt, bg2)


def _dispatch_body(xf_hbm, da_hbm, db_hbm, xs_hbm, xrows, ia, ib, sem):
    wid = lax.axis_index("s") * NC + lax.axis_index("c")
    base = wid * TPW
    pltpu.sync_copy(xf_hbm.at[pl.ds(base, TPW), :], xrows)
    pltpu.sync_copy(da_hbm.at[pl.ds(base, TPW)], ia)
    pltpu.sync_copy(db_hbm.at[pl.ds(base, TPW)], ib)
    c1 = pltpu.async_copy(xrows, xs_hbm.at[ia], sem)
    c2 = pltpu.async_copy(xrows, xs_hbm.at[ib], sem)
    c1.wait()
    c2.wait()


def _dispatch_call(xf, da, db):
    mesh = plsc.VectorSubcoreMesh(core_axis_name="c", subcore_axis_name="s")
    f = pl.kernel(
        _dispatch_body,
        out_type=jax.ShapeDtypeStruct((R, D), jnp.float32),
        mesh=mesh,
        scratch_types=[
            pltpu.VMEM((TPW, D), jnp.float32),
            pltpu.VMEM((TPW,), jnp.int32),
            pltpu.VMEM((TPW,), jnp.int32),
            pltpu.SemaphoreType.DMA,
        ],
    )
    return f(xf, da, db)


_NT = (((1,), (1,)), ((), ()))


def _ffn_body(be_ref, act_ref, xs_ref, w1_ref, b1_ref, w2_ref, b2_ref,
              ys_ref):
    g = pl.program_id(0)

    @pl.when(act_ref[g] > 0)
    def _():
        x = xs_ref[...].astype(jnp.bfloat16)
        h = lax.dot_general(x, w1_ref[0].astype(jnp.bfloat16), _NT,
                            preferred_element_type=jnp.float32)
        h = jnp.maximum(h + b1_ref[0], 0.0)
        y = lax.dot_general(h.astype(jnp.bfloat16),
                            w2_ref[0].astype(jnp.bfloat16), _NT,
                            preferred_element_type=jnp.float32)
        ys_ref[...] = y + b2_ref[0]


def _ffn_call(be, act, xs, w1, b1r, w2, b2r):
    grid_spec = pltpu.PrefetchScalarGridSpec(
        num_scalar_prefetch=2,
        grid=(GMAX,),
        in_specs=[
            pl.BlockSpec((B, D), lambda g, be, act: (g, 0)),
            pl.BlockSpec((1, D, D), lambda g, be, act: (be[g], 0, 0)),
            pl.BlockSpec((1, 1, D), lambda g, be, act: (be[g], 0, 0)),
            pl.BlockSpec((1, D, D), lambda g, be, act: (be[g], 0, 0)),
            pl.BlockSpec((1, 1, D), lambda g, be, act: (be[g], 0, 0)),
        ],
        out_specs=pl.BlockSpec((B, D), lambda g, be, act: (g, 0)),
    )
    return pl.pallas_call(
        _ffn_body,
        grid_spec=grid_spec,
        out_shape=jax.ShapeDtypeStruct((R, D), jnp.float32),
    )(be, act, xs, w1, b1r, w2, b2r)


CH = TPW // 2


def _combine_body(ys_hbm, da_hbm, db_hbm, ya_hbm, yb_hbm, bufa, bufb, ia, ib,
                  sem):
    wid = lax.axis_index("s") * NC + lax.axis_index("c")
    base = wid * TPW
    for chunk in range(2):
        tbase = base + chunk * CH
        pltpu.sync_copy(da_hbm.at[pl.ds(tbase, CH)], ia)
        pltpu.sync_copy(db_hbm.at[pl.ds(tbase, CH)], ib)
        ca = pltpu.async_copy(ys_hbm.at[ia], bufa, sem)
        cb = pltpu.async_copy(ys_hbm.at[ib], bufb, sem)
        ca.wait()
        cb.wait()
        pltpu.sync_copy(bufa, ya_hbm.at[pl.ds(tbase, CH), :])
        pltpu.sync_copy(bufb, yb_hbm.at[pl.ds(tbase, CH), :])


def _combine_call(ys, da, db):
    mesh = plsc.VectorSubcoreMesh(core_axis_name="c", subcore_axis_name="s")
    f = pl.kernel(
        _combine_body,
        out_type=(
            jax.ShapeDtypeStruct((NTOK, D), jnp.float32),
            jax.ShapeDtypeStruct((NTOK, D), jnp.float32),
        ),
        mesh=mesh,
        scratch_types=[
            pltpu.VMEM((CH, D), jnp.float32),
            pltpu.VMEM((CH, D), jnp.float32),
            pltpu.VMEM((CH,), jnp.int32),
            pltpu.VMEM((CH,), jnp.int32),
            pltpu.SemaphoreType.DMA,
        ],
    )
    return f(ys, da, db)


def _add_body(a_ref, b_ref, w_ref, o_ref):
    wv = w_ref[...]
    o_ref[...] = a_ref[...] * wv[:, 0:1] + b_ref[...] * wv[:, 1:2]


def _add_call(a, b, w2col):
    return pl.pallas_call(
        _add_body,
        grid=(4,),
        in_specs=[
            pl.BlockSpec((NTOK // 4, D), lambda g: (g, 0)),
            pl.BlockSpec((NTOK // 4, D), lambda g: (g, 0)),
            pl.BlockSpec((NTOK // 4, 2), lambda g: (g, 0)),
        ],
        out_specs=pl.BlockSpec((NTOK // 4, D), lambda g: (g, 0)),
        out_shape=jax.ShapeDtypeStruct((NTOK, D), jnp.float32),
    )(a, b, w2col)


def kernel(x, Wg, bg, W1, b1, W2, b2):
    xf = x.reshape(-1, D)
    dest_mat, w2col, be_row = _gating_call(xf, Wg.T, bg.reshape(1, E))

    d = jnp.transpose(dest_mat).reshape(-1)
    da, db = d[:NTOK], d[NTOK:]
    be = be_row[0, :GMAX]
    act = be_row[1, :GMAX]

    xs = _dispatch_call(xf, da, db)
    ys = _ffn_call(be, act, xs, W1, b1.reshape(E, 1, D), W2,
                   b2.reshape(E, 1, D))
    ya, yb = _combine_call(ys, da, db)
    y = _add_call(ya, yb, w2col)
    return y.reshape(x.shape)
